# Initial kernel scaffold; baseline (speedup 1.0000x reference)
import jax
import jax.numpy as jnp
from jax.experimental import pallas as pl


def kernel(token_embedding, idx, targets):
    raise NotImplementedError("write your pallas kernel here")


# speedup vs baseline: 1.7509x; 1.7509x over previous
import functools

import jax
import jax.numpy as jnp
from jax import lax
from jax.experimental import pallas as pl
from jax.experimental.pallas import tpu as pltpu
from jax.experimental.pallas import tpu_sc as plsc

V = 1000
N_TOK = 20480
NC = 2
NS = 16
NW = NC * NS
BPW = N_TOK // NW
CHUNK = 64
NCHUNK = BPW // CHUNK


def _lse_body(tab_ref, lse_ref):
    x = tab_ref[...]
    m = jnp.max(x, axis=1)
    s = jnp.sum(jnp.exp(x - m[:, None]), axis=1)
    lse_ref[...] = jnp.log(s) + m


def _row_lse(table):
    return pl.pallas_call(
        _lse_body,
        out_shape=jax.ShapeDtypeStruct((V,), jnp.float32),
    )(table)


def _sc_body(table_hbm, idx_hbm, tgt_hbm, lse_hbm,
             out_hbm, part_hbm,
             idx_v, tgt_v, lse_v, rows_v, part_v, sem):
    wid = lax.axis_index("s") * NC + lax.axis_index("c")
    base = wid * BPW

    pltpu.sync_copy(idx_hbm.at[pl.ds(base, BPW)], idx_v)
    pltpu.sync_copy(tgt_hbm.at[pl.ds(base, BPW)], tgt_v)
    pltpu.sync_copy(lse_hbm, lse_v)

    def chunk(c, acc):
        cbase = c * CHUNK
        idx_sl = idx_v.at[pl.ds(cbase, CHUNK)]
        pltpu.async_copy(table_hbm.at[idx_sl], rows_v, sem).wait()
        pltpu.sync_copy(rows_v, out_hbm.at[pl.ds(base + cbase, CHUNK)])
        for j in range(CHUNK // 16):
            off = cbase + j * 16
            rid = lax.iota(jnp.int32, 16) + j * 16
            tgt16 = tgt_v[pl.ds(off, 16)]
            idx16 = idx_v[pl.ds(off, 16)]
            tv = plsc.load_gather(rows_v, [rid, tgt16])
            lv = plsc.load_gather(lse_v, [idx16])
            acc = acc + (lv - tv)
        return acc

    acc = lax.fori_loop(0, NCHUNK, chunk, jnp.zeros((16,), jnp.float32))
    part_v[...] = acc
    pltpu.sync_copy(part_v, part_hbm.at[wid])


@functools.partial(
    pl.kernel,
    out_type=(jax.ShapeDtypeStruct((N_TOK, V), jnp.float32),
              jax.ShapeDtypeStruct((NW, 16), jnp.float32)),
    mesh=plsc.VectorSubcoreMesh(core_axis_name="c", subcore_axis_name="s"),
    scratch_types=[
        pltpu.VMEM((BPW,), jnp.int32),
        pltpu.VMEM((BPW,), jnp.int32),
        pltpu.VMEM((V,), jnp.float32),
        pltpu.VMEM((CHUNK, V), jnp.float32),
        pltpu.VMEM((16,), jnp.float32),
        pltpu.SemaphoreType.DMA,
    ],
    compiler_params=pltpu.CompilerParams(use_tc_tiling_on_sc=False,
                                         needs_layout_passes=False),
)
def _sc_gather(table_hbm, idx_hbm, tgt_hbm, lse_hbm, out_hbm, part_hbm,
               idx_v, tgt_v, lse_v, rows_v, part_v, sem):
    _sc_body(table_hbm, idx_hbm, tgt_hbm, lse_hbm, out_hbm, part_hbm,
             idx_v, tgt_v, lse_v, rows_v, part_v, sem)


def kernel(token_embedding, idx, targets):
    Bb, Tt = idx.shape
    lse = _row_lse(token_embedding)
    logits_flat, partials = _sc_gather(
        token_embedding, idx.reshape(-1), targets.reshape(-1), lse)
    loss = jnp.sum(partials) / (Bb * Tt)
    return logits_flat.reshape(Bb, Tt, V), loss
